# baseline (speedup 1.0000x reference)
import jax
import jax.numpy as jnp
from jax.experimental import pallas as pl
from jax.experimental.pallas import tpu as pltpu

_TB = 1024
_S = 10


def _body(a_ref, b_ref, c_ref, ws0_ref, wn0_ref, ws1_ref, wn1_ref,
          out_ref, h0a_ref, accb_ref, m1a_ref, m1b_ref):
    s = pl.program_id(1)
    f32 = jnp.float32
    relu = jax.nn.relu
    b = b_ref[...]
    c = c_ref[...]
    ws0 = ws0_ref[...]
    wn0 = wn0_ref[...]
    bs = relu(jnp.dot(b, ws0, preferred_element_type=f32))
    cs = relu(jnp.dot(c, wn0, preferred_element_type=f32))

    @pl.when(s == 0)
    def _():
        a = a_ref[...]
        h0a_ref[...] = relu(jnp.dot(a, ws0, preferred_element_type=f32))
        accb_ref[...] = b
        m1a_ref[...] = bs
        m1b_ref[...] = cs

    @pl.when(s > 0)
    def _():
        accb_ref[...] += b
        m1a_ref[...] += bs
        m1b_ref[...] += cs

    @pl.when(s == _S - 1)
    def _():
        inv = f32(1.0 / _S)
        mean_b = accb_ref[...] * inv
        h0b = relu(jnp.dot(mean_b, wn0, preferred_element_type=f32))
        h0a = h0a_ref[...]
        m1a = m1a_ref[...] * inv
        m1b = m1b_ref[...] * inv
        ws1 = ws1_ref[...]
        wn1 = wn1_ref[...]
        out_ref[:, :128] = (jnp.dot(h0a, ws1[:128], preferred_element_type=f32)
                            + jnp.dot(h0b, ws1[128:], preferred_element_type=f32))
        out_ref[:, 128:] = (jnp.dot(m1a, wn1[:128], preferred_element_type=f32)
                            + jnp.dot(m1b, wn1[128:], preferred_element_type=f32))


def kernel(input_features, W_self_0, W_neigh_0, W_self_1, W_neigh_1):
    n, _, f = input_features.shape
    d1 = W_self_0.shape[1]
    d2 = W_self_1.shape[1]
    tb = _TB
    grid = (n // tb, _S)
    x4 = input_features.reshape(n, input_features.shape[1], 1, f)
    a_spec = pl.BlockSpec((tb, None, None, f), lambda i, s: (i, 0, 0, 0))
    b_spec = pl.BlockSpec((tb, None, None, f), lambda i, s: (i, 1 + s, 0, 0))
    c_spec = pl.BlockSpec((tb, None, None, f), lambda i, s: (i, 1 + _S + s, 0, 0))
    w0_spec = pl.BlockSpec((f, d1), lambda i, s: (0, 0))
    w1_spec = pl.BlockSpec((2 * d1, d2), lambda i, s: (0, 0))
    out_spec = pl.BlockSpec((tb, 2 * d2), lambda i, s: (i, 0))
    return pl.pallas_call(
        _body,
        grid=grid,
        in_specs=[a_spec, b_spec, c_spec, w0_spec, w0_spec, w1_spec, w1_spec],
        out_specs=out_spec,
        out_shape=jax.ShapeDtypeStruct((n, 2 * d2), jnp.float32),
        scratch_shapes=[pltpu.VMEM((tb, d1), jnp.float32) for _ in range(4)],
    )(x4, x4, x4, W_self_0, W_neigh_0, W_self_1, W_neigh_1)
